# Initial kernel scaffold; baseline (speedup 1.0000x reference)
import jax
import jax.numpy as jnp
from jax.experimental import pallas as pl


def kernel(x, edge_index, edge_attr, W1, att_src1, att_dst1, b1, W2, att_src2, att_dst2, b2, W_out, b_out):
    raise NotImplementedError("write your pallas kernel here")


# speedup vs baseline: 41.5051x; 41.5051x over previous
import functools

import jax
import jax.numpy as jnp
from jax import lax
from jax.experimental import pallas as pl
from jax.experimental.pallas import tpu as pltpu
from jax.experimental.pallas import tpu_sc as plsc

_NC, _NS = 2, 16
_BLK = 2000


def _lrelu(v):
    return jnp.where(v > 0, v, 0.2 * v)


def _prep1_body(x_ref, w_ref, av_ref, dv_ref,
                hl_ref, as_ref, ad_ref, ma_ref, md_ref):
    i = pl.program_id(0)
    hl = jnp.dot(x_ref[...], w_ref[...], preferred_element_type=jnp.float32)
    hl_ref[...] = hl
    a = jnp.sum(hl * av_ref[...], axis=1, keepdims=True)
    d = jnp.sum(hl * dv_ref[...], axis=1, keepdims=True)
    as_ref[...] = a
    ad_ref[...] = d
    _acc_max(i, ma_ref, md_ref, a, d)


def _acc_max(i, ma_ref, md_ref, a, d):
    bm_a = jnp.max(a).reshape(1, 1)
    bm_d = jnp.max(d).reshape(1, 1)

    @pl.when(i == 0)
    def _():
        ma_ref[...] = bm_a
        md_ref[...] = bm_d

    @pl.when(i > 0)
    def _():
        ma_ref[...] = jnp.maximum(ma_ref[...], bm_a)
        md_ref[...] = jnp.maximum(md_ref[...], bm_d)


def _prep1(x, W1, av, dv):
    n = x.shape[0]
    g = n // _BLK
    return pl.pallas_call(
        _prep1_body,
        grid=(g,),
        in_specs=[
            pl.BlockSpec((_BLK, 7), lambda i: (i, 0)),
            pl.BlockSpec((7, 16), lambda i: (0, 0)),
            pl.BlockSpec((1, 16), lambda i: (0, 0)),
            pl.BlockSpec((1, 16), lambda i: (0, 0)),
        ],
        out_specs=[
            pl.BlockSpec((_BLK, 16), lambda i: (i, 0)),
            pl.BlockSpec((_BLK, 1), lambda i: (i, 0)),
            pl.BlockSpec((_BLK, 1), lambda i: (i, 0)),
            pl.BlockSpec((1, 1), lambda i: (0, 0)),
            pl.BlockSpec((1, 1), lambda i: (0, 0)),
        ],
        out_shape=[
            jax.ShapeDtypeStruct((n, 16), jnp.float32),
            jax.ShapeDtypeStruct((n, 1), jnp.float32),
            jax.ShapeDtypeStruct((n, 1), jnp.float32),
            jax.ShapeDtypeStruct((1, 1), jnp.float32),
            jax.ShapeDtypeStruct((1, 1), jnp.float32),
        ],
    )(x, W1, av, dv)


def _mid_body(acc_ref, ss_ref, hl_ref, as_ref, ad_ref, sh_ref, b_ref,
              w_ref, av_ref, dv_ref,
              hlo_ref, as2_ref, ad2_ref, ma_ref, md_ref):
    i = pl.program_id(0)
    ex = jnp.exp(_lrelu(as_ref[...] + ad_ref[...]) - sh_ref[0, 0])
    acc = acc_ref[0] + acc_ref[1] + ex * hl_ref[...]
    ss = ss_ref[0] + ss_ref[1] + ex + 1e-16
    h1 = jnp.maximum(acc / ss + b_ref[...], 0.0)
    hl2 = jnp.dot(h1, w_ref[...], preferred_element_type=jnp.float32)
    a = jnp.sum(hl2 * av_ref[...], axis=1, keepdims=True)
    d = jnp.sum(hl2 * dv_ref[...], axis=1, keepdims=True)
    hlo_ref[0] = hl2[:, :16]
    hlo_ref[1] = hl2[:, 16:]
    as2_ref[...] = a
    ad2_ref[...] = d
    _acc_max(i, ma_ref, md_ref, a, d)


def _mid(acc1, ssum1, hl1, as1, ad1, shift1, b1, W2, av2, dv2):
    n = hl1.shape[0]
    g = n // _BLK
    return pl.pallas_call(
        _mid_body,
        grid=(g,),
        in_specs=[
            pl.BlockSpec((2, _BLK, 16), lambda i: (0, i, 0)),
            pl.BlockSpec((2, _BLK, 1), lambda i: (0, i, 0)),
            pl.BlockSpec((_BLK, 16), lambda i: (i, 0)),
            pl.BlockSpec((_BLK, 1), lambda i: (i, 0)),
            pl.BlockSpec((_BLK, 1), lambda i: (i, 0)),
            pl.BlockSpec((1, 1), lambda i: (0, 0)),
            pl.BlockSpec((1, 16), lambda i: (0, 0)),
            pl.BlockSpec((16, 32), lambda i: (0, 0)),
            pl.BlockSpec((1, 32), lambda i: (0, 0)),
            pl.BlockSpec((1, 32), lambda i: (0, 0)),
        ],
        out_specs=[
            pl.BlockSpec((2, _BLK, 16), lambda i: (0, i, 0)),
            pl.BlockSpec((_BLK, 1), lambda i: (i, 0)),
            pl.BlockSpec((_BLK, 1), lambda i: (i, 0)),
            pl.BlockSpec((1, 1), lambda i: (0, 0)),
            pl.BlockSpec((1, 1), lambda i: (0, 0)),
        ],
        out_shape=[
            jax.ShapeDtypeStruct((2, n, 16), jnp.float32),
            jax.ShapeDtypeStruct((n, 1), jnp.float32),
            jax.ShapeDtypeStruct((n, 1), jnp.float32),
            jax.ShapeDtypeStruct((1, 1), jnp.float32),
            jax.ShapeDtypeStruct((1, 1), jnp.float32),
        ],
    )(acc1, ssum1, hl1, as1, ad1, shift1, b1, W2, av2, dv2)


def _fin_body(acc_ref, ss_ref, hl_ref, as_ref, ad_ref, sh_ref, b_ref,
              w_ref, bo_ref, out_ref):
    ex = jnp.exp(_lrelu(as_ref[...] + ad_ref[...]) - sh_ref[0, 0])
    ss = ss_ref[...] + ex + 1e-16
    h2a = jnp.maximum((acc_ref[0] + ex * hl_ref[0]) / ss + b_ref[:, :16], 0.0)
    h2b = jnp.maximum((acc_ref[1] + ex * hl_ref[1]) / ss + b_ref[:, 16:], 0.0)
    h2 = jnp.concatenate([h2a, h2b], axis=1)
    out_ref[...] = (
        jnp.dot(h2, w_ref[...], preferred_element_type=jnp.float32)
        + bo_ref[...]
    )


def _fin(acc2, ssum2, hl2, as2, ad2, shift2, b2, W_out, b_out):
    n = as2.shape[0]
    g = n // _BLK
    return pl.pallas_call(
        _fin_body,
        grid=(g,),
        in_specs=[
            pl.BlockSpec((2, _BLK, 16), lambda i: (0, i, 0)),
            pl.BlockSpec((_BLK, 1), lambda i: (i, 0)),
            pl.BlockSpec((2, _BLK, 16), lambda i: (0, i, 0)),
            pl.BlockSpec((_BLK, 1), lambda i: (i, 0)),
            pl.BlockSpec((_BLK, 1), lambda i: (i, 0)),
            pl.BlockSpec((1, 1), lambda i: (0, 0)),
            pl.BlockSpec((1, 32), lambda i: (0, 0)),
            pl.BlockSpec((32, 4), lambda i: (0, 0)),
            pl.BlockSpec((1, 4), lambda i: (0, 0)),
        ],
        out_specs=pl.BlockSpec((_BLK, 4), lambda i: (i, 0)),
        out_shape=jax.ShapeDtypeStruct((n, 4), jnp.float32),
    )(acc2, ssum2, hl2, as2, ad2, shift2, b2, W_out, b_out)


def _edge_pass(src, dst, table, asn, adn, shift8, n, channel_split):
    e = src.shape[0]
    per = e // _NS if channel_split else e // (_NC * _NS)
    nfull, tail = divmod(per, 128)
    assert tail % 8 == 0
    assert n % 160 == 0 and n % 80 == 0
    rpt = n // _NS
    n10 = n // 10

    mesh = plsc.VectorSubcoreMesh(core_axis_name="c", subcore_axis_name="s")

    zc = 125
    z1 = 2000

    def body(src_h, dst_h, tab_h, asn_h, adn_h, sh_h, acc_o, ss_o,
             sidx, sidx2, didx, asv, adv, exv, rows,
             sidxT, sidx2T, didxT, asvT, advT, exvT, rowsT,
             shiftv, zrow, z1d, acc_sh, ssum_sh, as_sh, ad_sh,
             semA, semB, semC):
        cid = lax.axis_index("c")
        sid = lax.axis_index("s")

        def zr(i, c):
            zrow[i, :] = jnp.zeros((16,), jnp.float32)
            return c
        lax.fori_loop(0, zc, zr, 0)

        def zo(i, c):
            z1d[pl.ds(i * 16, 16)] = jnp.zeros((16,), jnp.float32)
            return c
        lax.fori_loop(0, z1 // 16, zo, 0)

        def za(k, c):
            pltpu.sync_copy(zrow, acc_sh.at[pl.ds(sid * rpt + k * zc, zc)])
            return c
        lax.fori_loop(0, rpt // zc, za, 0)

        @pl.when(sid < 10)
        def _():
            off = sid * n10
            for k in range(5):
                pltpu.sync_copy(z1d, ssum_sh.at[pl.ds(off + k * z1, z1)])
            pltpu.sync_copy(asn_h.at[pl.ds(off, n10)], as_sh.at[pl.ds(off, n10)])
            pltpu.sync_copy(adn_h.at[pl.ds(off, n10)], ad_sh.at[pl.ds(off, n10)])

        pltpu.sync_copy(sh_h, shiftv)
        plsc.subcore_barrier()

        s = shiftv[pl.ds(0, 16)][0]
        base = sid * per if channel_split else (cid * _NS + sid) * per
        ioff = cid * n

        def do_block(off, nb, bufs):
            sidx_b, sidx2_b, didx_b, asv_b, adv_b, exv_b, rows_b = bufs
            pltpu.sync_copy(src_h.at[pl.ds(off, nb)], sidx_b)
            pltpu.sync_copy(dst_h.at[pl.ds(off, nb)], didx_b)
            if channel_split:
                for g in range(nb // 16):
                    sidx2_b[pl.ds(g * 16, 16)] = (
                        sidx_b[pl.ds(g * 16, 16)] + ioff)
                row_idx = sidx2_b
            else:
                row_idx = sidx_b
            c1 = pltpu.async_copy(as_sh.at[sidx_b], asv_b, semA)
            c2 = pltpu.async_copy(ad_sh.at[didx_b], adv_b, semB)
            c3 = pltpu.async_copy(tab_h.at[row_idx], rows_b, semC)
            c1.wait()
            c2.wait()
            c3.wait()
            for g in range(nb // 16):
                al = asv_b[pl.ds(g * 16, 16)] + adv_b[pl.ds(g * 16, 16)]
                al = jnp.where(al > 0, al, 0.2 * al)
                exv_b[pl.ds(g * 16, 16)] = jnp.exp(al - s)

            def scale(g, c):
                ev = exv_b[pl.ds(g * 16, 16)]
                b0 = g * 16
                for j in range(16):
                    rows_b[b0 + j, :] = rows_b[b0 + j, :] * ev[j]
                return c
            lax.fori_loop(0, nb // 16, scale, 0)
            pltpu.sync_copy(rows_b, acc_sh.at[didx_b], add=True)
            if channel_split:
                @pl.when(cid == 0)
                def _():
                    pltpu.sync_copy(exv_b, ssum_sh.at[didx_b], add=True)
            else:
                pltpu.sync_copy(exv_b, ssum_sh.at[didx_b], add=True)

        full_bufs = (sidx, sidx2, didx, asv, adv, exv, rows)
        tail_bufs = (sidxT, sidx2T, didxT, asvT, advT, exvT, rowsT)

        def fb(i, c):
            do_block(base + i * 128, 128, full_bufs)
            return c
        lax.fori_loop(0, nfull, fb, 0)
        if tail:
            do_block(base + nfull * 128, tail, tail_bufs)

        plsc.subcore_barrier()

        @pl.when(sid < 10)
        def _():
            ro = sid * n10
            pltpu.sync_copy(acc_sh.at[pl.ds(ro, n10)],
                            acc_o.at[pl.ds(cid * n + ro, n10)])

        pred = sid < 10
        if channel_split:
            pred = jnp.logical_and(pred, cid == 0)

        @pl.when(pred)
        def _():
            off = sid * n10
            pltpu.sync_copy(ssum_sh.at[pl.ds(off, n10)],
                            ss_o.at[pl.ds(cid * n + off, n10)])

    tail_n = tail if tail else 8
    run = pl.kernel(
        body,
        out_type=[
            jax.ShapeDtypeStruct((2 * n, 16), jnp.float32),
            jax.ShapeDtypeStruct((2 * n,), jnp.float32),
        ],
        mesh=mesh,
        scratch_types=[
            pltpu.VMEM((128,), jnp.int32),
            pltpu.VMEM((128,), jnp.int32),
            pltpu.VMEM((128,), jnp.int32),
            pltpu.VMEM((128,), jnp.float32),
            pltpu.VMEM((128,), jnp.float32),
            pltpu.VMEM((128,), jnp.float32),
            pltpu.VMEM((128, 16), jnp.float32),
            pltpu.VMEM((tail_n,), jnp.int32),
            pltpu.VMEM((tail_n,), jnp.int32),
            pltpu.VMEM((tail_n,), jnp.int32),
            pltpu.VMEM((tail_n,), jnp.float32),
            pltpu.VMEM((tail_n,), jnp.float32),
            pltpu.VMEM((tail_n,), jnp.float32),
            pltpu.VMEM((tail_n, 16), jnp.float32),
            pltpu.VMEM((16,), jnp.float32),
            pltpu.VMEM((zc, 16), jnp.float32),
            pltpu.VMEM((z1,), jnp.float32),
            pltpu.VMEM_SHARED((n, 16), jnp.float32),
            pltpu.VMEM_SHARED((n,), jnp.float32),
            pltpu.VMEM_SHARED((n,), jnp.float32),
            pltpu.VMEM_SHARED((n,), jnp.float32),
            pltpu.SemaphoreType.DMA,
            pltpu.SemaphoreType.DMA,
            pltpu.SemaphoreType.DMA,
        ],
        compiler_params=pltpu.CompilerParams(use_tc_tiling_on_sc=False),
    )
    return run(src, dst, table, asn, adn, shift8)


def _shift_arrays(ma, md):
    c = ma[0, 0] + md[0, 0]
    s = jnp.where(c > 0, c, 0.2 * c)
    return s.reshape(1, 1), jnp.broadcast_to(s.reshape(1), (16,))


def kernel(x, edge_index, edge_attr, W1, att_src1, att_dst1, b1,
           W2, att_src2, att_dst2, b2, W_out, b_out):
    n = x.shape[0]
    src = edge_index[0]
    dst = edge_index[1]

    hl1, as1, ad1, ma1, md1 = _prep1(
        x, W1, att_src1.reshape(1, 16), att_dst1.reshape(1, 16))
    sh1_tc, sh1_sc = _shift_arrays(ma1, md1)

    acc1, ssum1 = _edge_pass(
        src, dst, hl1, as1.reshape(n), ad1.reshape(n), sh1_sc, n,
        channel_split=False)

    hl2, as2, ad2, ma2, md2 = _mid(
        acc1.reshape(2, n, 16), ssum1.reshape(2, n, 1), hl1, as1, ad1,
        sh1_tc, b1.reshape(1, 16), W2,
        att_src2.reshape(1, 32), att_dst2.reshape(1, 32))
    sh2_tc, sh2_sc = _shift_arrays(ma2, md2)

    acc2, ssum2 = _edge_pass(
        src, dst, hl2.reshape(2 * n, 16), as2.reshape(n), ad2.reshape(n),
        sh2_sc, n, channel_split=True)

    return _fin(acc2.reshape(2, n, 16), ssum2[:n].reshape(n, 1), hl2,
                as2, ad2, sh2_tc, b2.reshape(1, 32), W_out,
                b_out.reshape(1, 4))


# speedup vs baseline: 61.0548x; 1.4710x over previous
import functools

import jax
import jax.numpy as jnp
from jax import lax
from jax.experimental import pallas as pl
from jax.experimental.pallas import tpu as pltpu
from jax.experimental.pallas import tpu_sc as plsc

_NC, _NS = 2, 16
_BLK = 2000


def _lrelu(v):
    return jnp.where(v > 0, v, 0.2 * v)


def _prep1_body(x_ref, w_ref, av_ref, dv_ref,
                hl_ref, as_ref, ad_ref, ma_ref, md_ref):
    i = pl.program_id(0)
    hl = jnp.dot(x_ref[...], w_ref[...], preferred_element_type=jnp.float32)
    hl_ref[...] = hl
    a = jnp.sum(hl * av_ref[...], axis=1, keepdims=True)
    d = jnp.sum(hl * dv_ref[...], axis=1, keepdims=True)
    as_ref[...] = a
    ad_ref[...] = d
    _acc_max(i, ma_ref, md_ref, a, d)


def _acc_max(i, ma_ref, md_ref, a, d):
    bm_a = jnp.max(a).reshape(1, 1)
    bm_d = jnp.max(d).reshape(1, 1)

    @pl.when(i == 0)
    def _():
        ma_ref[...] = bm_a
        md_ref[...] = bm_d

    @pl.when(i > 0)
    def _():
        ma_ref[...] = jnp.maximum(ma_ref[...], bm_a)
        md_ref[...] = jnp.maximum(md_ref[...], bm_d)


def _prep1(x, W1, av, dv):
    n = x.shape[0]
    g = n // _BLK
    return pl.pallas_call(
        _prep1_body,
        grid=(g,),
        in_specs=[
            pl.BlockSpec((_BLK, 7), lambda i: (i, 0)),
            pl.BlockSpec((7, 16), lambda i: (0, 0)),
            pl.BlockSpec((1, 16), lambda i: (0, 0)),
            pl.BlockSpec((1, 16), lambda i: (0, 0)),
        ],
        out_specs=[
            pl.BlockSpec((_BLK, 16), lambda i: (i, 0)),
            pl.BlockSpec((_BLK, 1), lambda i: (i, 0)),
            pl.BlockSpec((_BLK, 1), lambda i: (i, 0)),
            pl.BlockSpec((1, 1), lambda i: (0, 0)),
            pl.BlockSpec((1, 1), lambda i: (0, 0)),
        ],
        out_shape=[
            jax.ShapeDtypeStruct((n, 16), jnp.float32),
            jax.ShapeDtypeStruct((n, 1), jnp.float32),
            jax.ShapeDtypeStruct((n, 1), jnp.float32),
            jax.ShapeDtypeStruct((1, 1), jnp.float32),
            jax.ShapeDtypeStruct((1, 1), jnp.float32),
        ],
    )(x, W1, av, dv)


def _mid_body(acc_ref, ss_ref, hl_ref, as_ref, ad_ref, sh_ref, b_ref,
              w_ref, av_ref, dv_ref,
              hlo_ref, as2_ref, ad2_ref, ma_ref, md_ref):
    i = pl.program_id(0)
    ex = jnp.exp(_lrelu(as_ref[...] + ad_ref[...]) - sh_ref[0, 0])
    acc = acc_ref[0] + acc_ref[1] + ex * hl_ref[...]
    ss = ss_ref[0] + ss_ref[1] + ex + 1e-16
    h1 = jnp.maximum(acc / ss + b_ref[...], 0.0)
    hl2 = jnp.dot(h1, w_ref[...], preferred_element_type=jnp.float32)
    a = jnp.sum(hl2 * av_ref[...], axis=1, keepdims=True)
    d = jnp.sum(hl2 * dv_ref[...], axis=1, keepdims=True)
    hlo_ref[0] = hl2[:, :16]
    hlo_ref[1] = hl2[:, 16:]
    as2_ref[...] = a
    ad2_ref[...] = d
    _acc_max(i, ma_ref, md_ref, a, d)


def _mid(acc1, ssum1, hl1, as1, ad1, shift1, b1, W2, av2, dv2):
    n = hl1.shape[0]
    g = n // _BLK
    return pl.pallas_call(
        _mid_body,
        grid=(g,),
        in_specs=[
            pl.BlockSpec((2, _BLK, 16), lambda i: (0, i, 0)),
            pl.BlockSpec((2, _BLK, 1), lambda i: (0, i, 0)),
            pl.BlockSpec((_BLK, 16), lambda i: (i, 0)),
            pl.BlockSpec((_BLK, 1), lambda i: (i, 0)),
            pl.BlockSpec((_BLK, 1), lambda i: (i, 0)),
            pl.BlockSpec((1, 1), lambda i: (0, 0)),
            pl.BlockSpec((1, 16), lambda i: (0, 0)),
            pl.BlockSpec((16, 32), lambda i: (0, 0)),
            pl.BlockSpec((1, 32), lambda i: (0, 0)),
            pl.BlockSpec((1, 32), lambda i: (0, 0)),
        ],
        out_specs=[
            pl.BlockSpec((2, _BLK, 16), lambda i: (0, i, 0)),
            pl.BlockSpec((_BLK, 1), lambda i: (i, 0)),
            pl.BlockSpec((_BLK, 1), lambda i: (i, 0)),
            pl.BlockSpec((1, 1), lambda i: (0, 0)),
            pl.BlockSpec((1, 1), lambda i: (0, 0)),
        ],
        out_shape=[
            jax.ShapeDtypeStruct((2, n, 16), jnp.float32),
            jax.ShapeDtypeStruct((n, 1), jnp.float32),
            jax.ShapeDtypeStruct((n, 1), jnp.float32),
            jax.ShapeDtypeStruct((1, 1), jnp.float32),
            jax.ShapeDtypeStruct((1, 1), jnp.float32),
        ],
    )(acc1, ssum1, hl1, as1, ad1, shift1, b1, W2, av2, dv2)


def _fin_body(acc_ref, ss_ref, hl_ref, as_ref, ad_ref, sh_ref, b_ref,
              w_ref, bo_ref, out_ref):
    ex = jnp.exp(_lrelu(as_ref[...] + ad_ref[...]) - sh_ref[0, 0])
    ss = ss_ref[...] + ex + 1e-16
    h2a = jnp.maximum((acc_ref[0] + ex * hl_ref[0]) / ss + b_ref[:, :16], 0.0)
    h2b = jnp.maximum((acc_ref[1] + ex * hl_ref[1]) / ss + b_ref[:, 16:], 0.0)
    h2 = jnp.concatenate([h2a, h2b], axis=1)
    out_ref[...] = (
        jnp.dot(h2, w_ref[...], preferred_element_type=jnp.float32)
        + bo_ref[...]
    )


def _fin(acc2, ssum2, hl2, as2, ad2, shift2, b2, W_out, b_out):
    n = as2.shape[0]
    g = n // _BLK
    return pl.pallas_call(
        _fin_body,
        grid=(g,),
        in_specs=[
            pl.BlockSpec((2, _BLK, 16), lambda i: (0, i, 0)),
            pl.BlockSpec((_BLK, 1), lambda i: (i, 0)),
            pl.BlockSpec((2, _BLK, 16), lambda i: (0, i, 0)),
            pl.BlockSpec((_BLK, 1), lambda i: (i, 0)),
            pl.BlockSpec((_BLK, 1), lambda i: (i, 0)),
            pl.BlockSpec((1, 1), lambda i: (0, 0)),
            pl.BlockSpec((1, 32), lambda i: (0, 0)),
            pl.BlockSpec((32, 4), lambda i: (0, 0)),
            pl.BlockSpec((1, 4), lambda i: (0, 0)),
        ],
        out_specs=pl.BlockSpec((_BLK, 4), lambda i: (i, 0)),
        out_shape=jax.ShapeDtypeStruct((n, 4), jnp.float32),
    )(acc2, ssum2, hl2, as2, ad2, shift2, b2, W_out, b_out)


def _edge_pass(src, dst, table, asn, adn, shift8, n, channel_split):
    e = src.shape[0]
    per = e // _NS if channel_split else e // (_NC * _NS)
    nfull, tail = divmod(per, 128)
    assert tail % 8 == 0
    assert n % 160 == 0 and n % 80 == 0
    rpt = n // _NS
    n10 = n // 10

    mesh = plsc.VectorSubcoreMesh(core_axis_name="c", subcore_axis_name="s")

    zc = 125
    z1 = 2000

    def body(src_h, dst_h, tab_h, asn_h, adn_h, sh_h, acc_o, ss_o,
             sidx, sidx2, didx, asv, adv, exv, rows,
             sidxB, sidx2B, didxB, asvB, advB, exvB, rowsB,
             sidxT, sidx2T, didxT, asvT, advT, exvT, rowsT,
             shiftv, zrow, z1d, acc_sh, ssum_sh, as_sh, ad_sh,
             semA, semB, semC):
        cid = lax.axis_index("c")
        sid = lax.axis_index("s")

        def zr(i, c):
            zrow[i, :] = jnp.zeros((16,), jnp.float32)
            return c
        lax.fori_loop(0, zc, zr, 0)

        def zo(i, c):
            z1d[pl.ds(i * 16, 16)] = jnp.zeros((16,), jnp.float32)
            return c
        lax.fori_loop(0, z1 // 16, zo, 0)

        def za(k, c):
            pltpu.sync_copy(zrow, acc_sh.at[pl.ds(sid * rpt + k * zc, zc)])
            return c
        lax.fori_loop(0, rpt // zc, za, 0)

        @pl.when(sid < 10)
        def _():
            off = sid * n10
            for k in range(5):
                pltpu.sync_copy(z1d, ssum_sh.at[pl.ds(off + k * z1, z1)])
            pltpu.sync_copy(asn_h.at[pl.ds(off, n10)], as_sh.at[pl.ds(off, n10)])
            pltpu.sync_copy(adn_h.at[pl.ds(off, n10)], ad_sh.at[pl.ds(off, n10)])

        pltpu.sync_copy(sh_h, shiftv)
        plsc.subcore_barrier()

        s = shiftv[pl.ds(0, 16)][0]
        base = sid * per if channel_split else (cid * _NS + sid) * per
        ioff = cid * n

        def row_ref(bufs):
            return bufs[1] if channel_split else bufs[0]

        def idx_load(off, bufs, nb):
            sidx_b, sidx2_b, didx_b = bufs[0], bufs[1], bufs[2]
            pltpu.sync_copy(src_h.at[pl.ds(off, nb)], sidx_b)
            pltpu.sync_copy(dst_h.at[pl.ds(off, nb)], didx_b)
            if channel_split:
                for g in range(nb // 16):
                    sidx2_b[pl.ds(g * 16, 16)] = (
                        sidx_b[pl.ds(g * 16, 16)] + ioff)

        def fire_gathers(bufs):
            sidx_b, _, didx_b, asv_b, adv_b, _, rows_b = bufs
            pltpu.async_copy(as_sh.at[sidx_b], asv_b, semA)
            pltpu.async_copy(ad_sh.at[didx_b], adv_b, semB)
            pltpu.async_copy(tab_h.at[row_ref(bufs)], rows_b, semC)

        def wait_gathers(bufs):
            sidx_b, _, didx_b, asv_b, adv_b, _, rows_b = bufs
            pltpu.make_async_copy(as_sh.at[sidx_b], asv_b, semA).wait()
            pltpu.make_async_copy(ad_sh.at[didx_b], adv_b, semB).wait()
            pltpu.make_async_copy(tab_h.at[row_ref(bufs)], rows_b,
                                  semC).wait()

        def compute_scatter(bufs, nb):
            _, _, didx_b, asv_b, adv_b, exv_b, rows_b = bufs
            for g in range(nb // 16):
                al = asv_b[pl.ds(g * 16, 16)] + adv_b[pl.ds(g * 16, 16)]
                al = jnp.where(al > 0, al, 0.2 * al)
                exv_b[pl.ds(g * 16, 16)] = jnp.exp(al - s)

            def scale(g, c):
                ev = exv_b[pl.ds(g * 16, 16)]
                b0 = g * 16
                for j in range(16):
                    rows_b[b0 + j, :] = rows_b[b0 + j, :] * ev[j]
                return c
            lax.fori_loop(0, nb // 16, scale, 0)
            pltpu.sync_copy(rows_b, acc_sh.at[didx_b], add=True)
            if channel_split:
                @pl.when(cid == 0)
                def _():
                    pltpu.sync_copy(exv_b, ssum_sh.at[didx_b], add=True)
            else:
                pltpu.sync_copy(exv_b, ssum_sh.at[didx_b], add=True)

        def do_block(off, nb, bufs):
            idx_load(off, bufs, nb)
            fire_gathers(bufs)
            wait_gathers(bufs)
            compute_scatter(bufs, nb)

        bufs_a = (sidx, sidx2, didx, asv, adv, exv, rows)
        bufs_b = (sidxB, sidx2B, didxB, asvB, advB, exvB, rowsB)
        tail_bufs = (sidxT, sidx2T, didxT, asvT, advT, exvT, rowsT)

        nfull_even = nfull - (nfull % 2)
        npair = nfull_even // 2

        idx_load(base, bufs_a, 128)
        fire_gathers(bufs_a)

        def pbody(k, c):
            o0 = base + (2 * k) * 128
            idx_load(o0 + 128, bufs_b, 128)
            wait_gathers(bufs_a)
            fire_gathers(bufs_b)
            compute_scatter(bufs_a, 128)

            @pl.when(k < npair - 1)
            def _():
                idx_load(o0 + 256, bufs_a, 128)
                fire_gathers(bufs_a)
            wait_gathers(bufs_b)
            compute_scatter(bufs_b, 128)
            return c
        lax.fori_loop(0, npair, pbody, 0)

        for j in range(nfull - nfull_even):
            do_block(base + (nfull_even + j) * 128, 128, bufs_a)
        if tail:
            do_block(base + nfull * 128, tail, tail_bufs)

        plsc.subcore_barrier()

        @pl.when(sid < 10)
        def _():
            ro = sid * n10
            pltpu.sync_copy(acc_sh.at[pl.ds(ro, n10)],
                            acc_o.at[pl.ds(cid * n + ro, n10)])

        pred = sid < 10
        if channel_split:
            pred = jnp.logical_and(pred, cid == 0)

        @pl.when(pred)
        def _():
            off = sid * n10
            pltpu.sync_copy(ssum_sh.at[pl.ds(off, n10)],
                            ss_o.at[pl.ds(cid * n + off, n10)])

    tail_n = tail if tail else 8
    run = pl.kernel(
        body,
        out_type=[
            jax.ShapeDtypeStruct((2 * n, 16), jnp.float32),
            jax.ShapeDtypeStruct((2 * n,), jnp.float32),
        ],
        mesh=mesh,
        scratch_types=[
            pltpu.VMEM((128,), jnp.int32),
            pltpu.VMEM((128,), jnp.int32),
            pltpu.VMEM((128,), jnp.int32),
            pltpu.VMEM((128,), jnp.float32),
            pltpu.VMEM((128,), jnp.float32),
            pltpu.VMEM((128,), jnp.float32),
            pltpu.VMEM((128, 16), jnp.float32),
            pltpu.VMEM((128,), jnp.int32),
            pltpu.VMEM((128,), jnp.int32),
            pltpu.VMEM((128,), jnp.int32),
            pltpu.VMEM((128,), jnp.float32),
            pltpu.VMEM((128,), jnp.float32),
            pltpu.VMEM((128,), jnp.float32),
            pltpu.VMEM((128, 16), jnp.float32),
            pltpu.VMEM((tail_n,), jnp.int32),
            pltpu.VMEM((tail_n,), jnp.int32),
            pltpu.VMEM((tail_n,), jnp.int32),
            pltpu.VMEM((tail_n,), jnp.float32),
            pltpu.VMEM((tail_n,), jnp.float32),
            pltpu.VMEM((tail_n,), jnp.float32),
            pltpu.VMEM((tail_n, 16), jnp.float32),
            pltpu.VMEM((16,), jnp.float32),
            pltpu.VMEM((zc, 16), jnp.float32),
            pltpu.VMEM((z1,), jnp.float32),
            pltpu.VMEM_SHARED((n, 16), jnp.float32),
            pltpu.VMEM_SHARED((n,), jnp.float32),
            pltpu.VMEM_SHARED((n,), jnp.float32),
            pltpu.VMEM_SHARED((n,), jnp.float32),
            pltpu.SemaphoreType.DMA,
            pltpu.SemaphoreType.DMA,
            pltpu.SemaphoreType.DMA,
        ],
        compiler_params=pltpu.CompilerParams(use_tc_tiling_on_sc=False),
    )
    return run(src, dst, table, asn, adn, shift8)


def _shift_arrays(ma, md):
    c = ma[0, 0] + md[0, 0]
    s = jnp.where(c > 0, c, 0.2 * c)
    return s.reshape(1, 1), jnp.broadcast_to(s.reshape(1), (16,))


def kernel(x, edge_index, edge_attr, W1, att_src1, att_dst1, b1,
           W2, att_src2, att_dst2, b2, W_out, b_out):
    n = x.shape[0]
    src = edge_index[0]
    dst = edge_index[1]

    hl1, as1, ad1, ma1, md1 = _prep1(
        x, W1, att_src1.reshape(1, 16), att_dst1.reshape(1, 16))
    sh1_tc, sh1_sc = _shift_arrays(ma1, md1)

    acc1, ssum1 = _edge_pass(
        src, dst, hl1, as1.reshape(n), ad1.reshape(n), sh1_sc, n,
        channel_split=False)

    hl2, as2, ad2, ma2, md2 = _mid(
        acc1.reshape(2, n, 16), ssum1.reshape(2, n, 1), hl1, as1, ad1,
        sh1_tc, b1.reshape(1, 16), W2,
        att_src2.reshape(1, 32), att_dst2.reshape(1, 32))
    sh2_tc, sh2_sc = _shift_arrays(ma2, md2)

    acc2, ssum2 = _edge_pass(
        src, dst, hl2.reshape(2 * n, 16), as2.reshape(n), ad2.reshape(n),
        sh2_sc, n, channel_split=True)

    return _fin(acc2.reshape(2, n, 16), ssum2[:n].reshape(n, 1), hl2,
                as2, ad2, sh2_tc, b2.reshape(1, 32), W_out,
                b_out.reshape(1, 4))


# speedup vs baseline: 63.6032x; 1.0417x over previous
import functools

import jax
import jax.numpy as jnp
from jax import lax
from jax.experimental import pallas as pl
from jax.experimental.pallas import tpu as pltpu
from jax.experimental.pallas import tpu_sc as plsc

_NC, _NS = 2, 16
_BLK = 2000


def _lrelu(v):
    return jnp.where(v > 0, v, 0.2 * v)


def _prep1_body(x_ref, w_ref, av_ref, dv_ref,
                hl_ref, as_ref, ad_ref, ma_ref, md_ref):
    i = pl.program_id(0)
    hl = jnp.dot(x_ref[...], w_ref[...], preferred_element_type=jnp.float32)
    hl_ref[...] = hl
    a = jnp.sum(hl * av_ref[...], axis=1, keepdims=True)
    d = jnp.sum(hl * dv_ref[...], axis=1, keepdims=True)
    as_ref[...] = a
    ad_ref[...] = d
    _acc_max(i, ma_ref, md_ref, a, d)


def _acc_max(i, ma_ref, md_ref, a, d):
    bm_a = jnp.max(a).reshape(1, 1)
    bm_d = jnp.max(d).reshape(1, 1)

    @pl.when(i == 0)
    def _():
        ma_ref[...] = bm_a
        md_ref[...] = bm_d

    @pl.when(i > 0)
    def _():
        ma_ref[...] = jnp.maximum(ma_ref[...], bm_a)
        md_ref[...] = jnp.maximum(md_ref[...], bm_d)


def _prep1(x, W1, av, dv):
    n = x.shape[0]
    g = n // _BLK
    return pl.pallas_call(
        _prep1_body,
        grid=(g,),
        in_specs=[
            pl.BlockSpec((_BLK, 7), lambda i: (i, 0)),
            pl.BlockSpec((7, 16), lambda i: (0, 0)),
            pl.BlockSpec((1, 16), lambda i: (0, 0)),
            pl.BlockSpec((1, 16), lambda i: (0, 0)),
        ],
        out_specs=[
            pl.BlockSpec((_BLK, 16), lambda i: (i, 0)),
            pl.BlockSpec((_BLK, 1), lambda i: (i, 0)),
            pl.BlockSpec((_BLK, 1), lambda i: (i, 0)),
            pl.BlockSpec((1, 1), lambda i: (0, 0)),
            pl.BlockSpec((1, 1), lambda i: (0, 0)),
        ],
        out_shape=[
            jax.ShapeDtypeStruct((n, 16), jnp.float32),
            jax.ShapeDtypeStruct((n, 1), jnp.float32),
            jax.ShapeDtypeStruct((n, 1), jnp.float32),
            jax.ShapeDtypeStruct((1, 1), jnp.float32),
            jax.ShapeDtypeStruct((1, 1), jnp.float32),
        ],
    )(x, W1, av, dv)


def _mid_body(acc_ref, ss_ref, hl_ref, as_ref, ad_ref, sh_ref, b_ref,
              w_ref, av_ref, dv_ref,
              hlo_ref, as2_ref, ad2_ref, ma_ref, md_ref):
    i = pl.program_id(0)
    ex = jnp.exp(_lrelu(as_ref[...] + ad_ref[...]) - sh_ref[0, 0])
    acc = acc_ref[0] + acc_ref[1] + ex * hl_ref[...]
    ss = ss_ref[0] + ss_ref[1] + ex + 1e-16
    h1 = jnp.maximum(acc / ss + b_ref[...], 0.0)
    hl2 = jnp.dot(h1, w_ref[...], preferred_element_type=jnp.float32)
    a = jnp.sum(hl2 * av_ref[...], axis=1, keepdims=True)
    d = jnp.sum(hl2 * dv_ref[...], axis=1, keepdims=True)
    hlo_ref[0] = hl2[:, :16]
    hlo_ref[1] = hl2[:, 16:]
    as2_ref[...] = a
    ad2_ref[...] = d
    _acc_max(i, ma_ref, md_ref, a, d)


def _mid(acc1, ssum1, hl1, as1, ad1, shift1, b1, W2, av2, dv2):
    n = hl1.shape[0]
    g = n // _BLK
    return pl.pallas_call(
        _mid_body,
        grid=(g,),
        in_specs=[
            pl.BlockSpec((2, _BLK, 16), lambda i: (0, i, 0)),
            pl.BlockSpec((2, _BLK, 1), lambda i: (0, i, 0)),
            pl.BlockSpec((_BLK, 16), lambda i: (i, 0)),
            pl.BlockSpec((_BLK, 1), lambda i: (i, 0)),
            pl.BlockSpec((_BLK, 1), lambda i: (i, 0)),
            pl.BlockSpec((1, 1), lambda i: (0, 0)),
            pl.BlockSpec((1, 16), lambda i: (0, 0)),
            pl.BlockSpec((16, 32), lambda i: (0, 0)),
            pl.BlockSpec((1, 32), lambda i: (0, 0)),
            pl.BlockSpec((1, 32), lambda i: (0, 0)),
        ],
        out_specs=[
            pl.BlockSpec((2, _BLK, 16), lambda i: (0, i, 0)),
            pl.BlockSpec((_BLK, 1), lambda i: (i, 0)),
            pl.BlockSpec((_BLK, 1), lambda i: (i, 0)),
            pl.BlockSpec((1, 1), lambda i: (0, 0)),
            pl.BlockSpec((1, 1), lambda i: (0, 0)),
        ],
        out_shape=[
            jax.ShapeDtypeStruct((2, n, 16), jnp.float32),
            jax.ShapeDtypeStruct((n, 1), jnp.float32),
            jax.ShapeDtypeStruct((n, 1), jnp.float32),
            jax.ShapeDtypeStruct((1, 1), jnp.float32),
            jax.ShapeDtypeStruct((1, 1), jnp.float32),
        ],
    )(acc1, ssum1, hl1, as1, ad1, shift1, b1, W2, av2, dv2)


def _fin_body(acc_ref, ss_ref, hl_ref, as_ref, ad_ref, sh_ref, b_ref,
              w_ref, bo_ref, out_ref):
    ex = jnp.exp(_lrelu(as_ref[...] + ad_ref[...]) - sh_ref[0, 0])
    ss = ss_ref[...] + ex + 1e-16
    h2a = jnp.maximum((acc_ref[0] + ex * hl_ref[0]) / ss + b_ref[:, :16], 0.0)
    h2b = jnp.maximum((acc_ref[1] + ex * hl_ref[1]) / ss + b_ref[:, 16:], 0.0)
    h2 = jnp.concatenate([h2a, h2b], axis=1)
    out_ref[...] = (
        jnp.dot(h2, w_ref[...], preferred_element_type=jnp.float32)
        + bo_ref[...]
    )


def _fin(acc2, ssum2, hl2, as2, ad2, shift2, b2, W_out, b_out):
    n = as2.shape[0]
    g = n // _BLK
    return pl.pallas_call(
        _fin_body,
        grid=(g,),
        in_specs=[
            pl.BlockSpec((2, _BLK, 16), lambda i: (0, i, 0)),
            pl.BlockSpec((_BLK, 1), lambda i: (i, 0)),
            pl.BlockSpec((2, _BLK, 16), lambda i: (0, i, 0)),
            pl.BlockSpec((_BLK, 1), lambda i: (i, 0)),
            pl.BlockSpec((_BLK, 1), lambda i: (i, 0)),
            pl.BlockSpec((1, 1), lambda i: (0, 0)),
            pl.BlockSpec((1, 32), lambda i: (0, 0)),
            pl.BlockSpec((32, 4), lambda i: (0, 0)),
            pl.BlockSpec((1, 4), lambda i: (0, 0)),
        ],
        out_specs=pl.BlockSpec((_BLK, 4), lambda i: (i, 0)),
        out_shape=jax.ShapeDtypeStruct((n, 4), jnp.float32),
    )(acc2, ssum2, hl2, as2, ad2, shift2, b2, W_out, b_out)


def _edge_pass(src, dst, table, asn, adn, shift8, n, channel_split):
    e = src.shape[0]
    per = e // _NS if channel_split else e // (_NC * _NS)
    nfull, tail = divmod(per, 128)
    assert tail % 8 == 0
    assert n % 160 == 0 and n % 80 == 0
    rpt = n // _NS
    n10 = n // 10

    mesh = plsc.VectorSubcoreMesh(core_axis_name="c", subcore_axis_name="s")

    zc = 125
    z1 = 2000

    def body(src_h, dst_h, tab_h, asn_h, adn_h, sh_h, acc_o, ss_o,
             sidx, sidx2, didx, asv, adv, exv, rows,
             sidxB, sidx2B, didxB, asvB, advB, exvB, rowsB,
             sidxT, sidx2T, didxT, asvT, advT, exvT, rowsT,
             shiftv, zrow, z1d, acc_sh, ssum_sh, as_sh, ad_sh,
             semA, semB, semC, semSa, semUa, semSb, semUb):
        cid = lax.axis_index("c")
        sid = lax.axis_index("s")

        def zr(i, c):
            zrow[i, :] = jnp.zeros((16,), jnp.float32)
            return c
        lax.fori_loop(0, zc, zr, 0)

        def zo(i, c):
            z1d[pl.ds(i * 16, 16)] = jnp.zeros((16,), jnp.float32)
            return c
        lax.fori_loop(0, z1 // 16, zo, 0)

        def za(k, c):
            pltpu.sync_copy(zrow, acc_sh.at[pl.ds(sid * rpt + k * zc, zc)])
            return c
        lax.fori_loop(0, rpt // zc, za, 0)

        @pl.when(sid < 10)
        def _():
            off = sid * n10
            for k in range(5):
                pltpu.sync_copy(z1d, ssum_sh.at[pl.ds(off + k * z1, z1)])
            pltpu.sync_copy(asn_h.at[pl.ds(off, n10)], as_sh.at[pl.ds(off, n10)])
            pltpu.sync_copy(adn_h.at[pl.ds(off, n10)], ad_sh.at[pl.ds(off, n10)])

        pltpu.sync_copy(sh_h, shiftv)
        plsc.subcore_barrier()

        s = shiftv[pl.ds(0, 16)][0]
        base = sid * per if channel_split else (cid * _NS + sid) * per
        ioff = cid * n

        def row_ref(bufs):
            return bufs[1] if channel_split else bufs[0]

        def idx_load(off, bufs, nb):
            sidx_b, sidx2_b, didx_b = bufs[0], bufs[1], bufs[2]
            pltpu.sync_copy(src_h.at[pl.ds(off, nb)], sidx_b)
            pltpu.sync_copy(dst_h.at[pl.ds(off, nb)], didx_b)
            if channel_split:
                for g in range(nb // 16):
                    sidx2_b[pl.ds(g * 16, 16)] = (
                        sidx_b[pl.ds(g * 16, 16)] + ioff)

        def fire_gathers(bufs):
            sidx_b, _, didx_b, asv_b, adv_b, _, rows_b = bufs
            pltpu.async_copy(as_sh.at[sidx_b], asv_b, semA)
            pltpu.async_copy(ad_sh.at[didx_b], adv_b, semB)
            pltpu.async_copy(tab_h.at[row_ref(bufs)], rows_b, semC)

        def wait_gathers(bufs):
            sidx_b, _, didx_b, asv_b, adv_b, _, rows_b = bufs
            pltpu.make_async_copy(as_sh.at[sidx_b], asv_b, semA).wait()
            pltpu.make_async_copy(ad_sh.at[didx_b], adv_b, semB).wait()
            pltpu.make_async_copy(tab_h.at[row_ref(bufs)], rows_b,
                                  semC).wait()

        def compute_ex_scale(bufs, nb):
            _, _, didx_b, asv_b, adv_b, exv_b, rows_b = bufs
            for g in range(nb // 16):
                al = asv_b[pl.ds(g * 16, 16)] + adv_b[pl.ds(g * 16, 16)]
                al = jnp.where(al > 0, al, 0.2 * al)
                exv_b[pl.ds(g * 16, 16)] = jnp.exp(al - s)

            def scale(g, c):
                ev = exv_b[pl.ds(g * 16, 16)]
                b0 = g * 16
                for j in range(16):
                    rows_b[b0 + j, :] = rows_b[b0 + j, :] * ev[j]
                return c
            lax.fori_loop(0, nb // 16, scale, 0)

        def fire_scatter(bufs, semS, semU):
            _, _, didx_b, _, _, exv_b, rows_b = bufs
            pltpu.async_copy(rows_b, acc_sh.at[didx_b], semS, add=True)
            if channel_split:
                @pl.when(cid == 0)
                def _():
                    pltpu.async_copy(exv_b, ssum_sh.at[didx_b], semU,
                                     add=True)
            else:
                pltpu.async_copy(exv_b, ssum_sh.at[didx_b], semU, add=True)

        def drain_scatter(bufs, semS, semU):
            _, _, didx_b, _, _, exv_b, rows_b = bufs
            pltpu.make_async_copy(rows_b, acc_sh.at[didx_b], semS).wait()
            if channel_split:
                @pl.when(cid == 0)
                def _():
                    pltpu.make_async_copy(exv_b, ssum_sh.at[didx_b],
                                          semU).wait()
            else:
                pltpu.make_async_copy(exv_b, ssum_sh.at[didx_b],
                                      semU).wait()

        def do_block(off, nb, bufs):
            idx_load(off, bufs, nb)
            fire_gathers(bufs)
            wait_gathers(bufs)
            compute_ex_scale(bufs, nb)
            _, _, didx_b, _, _, exv_b, rows_b = bufs
            pltpu.sync_copy(rows_b, acc_sh.at[didx_b], add=True)
            if channel_split:
                @pl.when(cid == 0)
                def _():
                    pltpu.sync_copy(exv_b, ssum_sh.at[didx_b], add=True)
            else:
                pltpu.sync_copy(exv_b, ssum_sh.at[didx_b], add=True)

        bufs_a = (sidx, sidx2, didx, asv, adv, exv, rows)
        bufs_b = (sidxB, sidx2B, didxB, asvB, advB, exvB, rowsB)
        tail_bufs = (sidxT, sidx2T, didxT, asvT, advT, exvT, rowsT)

        nfull_even = nfull - (nfull % 2)
        npair = nfull_even // 2

        idx_load(base, bufs_a, 128)
        fire_gathers(bufs_a)

        def pbody(k, c):
            o0 = base + (2 * k) * 128

            @pl.when(k > 0)
            def _():
                drain_scatter(bufs_b, semSb, semUb)
            idx_load(o0 + 128, bufs_b, 128)
            wait_gathers(bufs_a)
            fire_gathers(bufs_b)
            compute_ex_scale(bufs_a, 128)
            fire_scatter(bufs_a, semSa, semUa)

            @pl.when(k < npair - 1)
            def _():
                drain_scatter(bufs_a, semSa, semUa)
                idx_load(o0 + 256, bufs_a, 128)
                fire_gathers(bufs_a)
            wait_gathers(bufs_b)
            compute_ex_scale(bufs_b, 128)
            fire_scatter(bufs_b, semSb, semUb)
            return c
        if npair:
            lax.fori_loop(0, npair, pbody, 0)
            drain_scatter(bufs_a, semSa, semUa)
            drain_scatter(bufs_b, semSb, semUb)

        for j in range(nfull - nfull_even):
            do_block(base + (nfull_even + j) * 128, 128, bufs_a)
        if tail:
            do_block(base + nfull * 128, tail, tail_bufs)

        plsc.subcore_barrier()

        @pl.when(sid < 10)
        def _():
            ro = sid * n10
            pltpu.sync_copy(acc_sh.at[pl.ds(ro, n10)],
                            acc_o.at[pl.ds(cid * n + ro, n10)])

        pred = sid < 10
        if channel_split:
            pred = jnp.logical_and(pred, cid == 0)

        @pl.when(pred)
        def _():
            off = sid * n10
            pltpu.sync_copy(ssum_sh.at[pl.ds(off, n10)],
                            ss_o.at[pl.ds(cid * n + off, n10)])

    tail_n = tail if tail else 8
    run = pl.kernel(
        body,
        out_type=[
            jax.ShapeDtypeStruct((2 * n, 16), jnp.float32),
            jax.ShapeDtypeStruct((2 * n,), jnp.float32),
        ],
        mesh=mesh,
        scratch_types=[
            pltpu.VMEM((128,), jnp.int32),
            pltpu.VMEM((128,), jnp.int32),
            pltpu.VMEM((128,), jnp.int32),
            pltpu.VMEM((128,), jnp.float32),
            pltpu.VMEM((128,), jnp.float32),
            pltpu.VMEM((128,), jnp.float32),
            pltpu.VMEM((128, 16), jnp.float32),
            pltpu.VMEM((128,), jnp.int32),
            pltpu.VMEM((128,), jnp.int32),
            pltpu.VMEM((128,), jnp.int32),
            pltpu.VMEM((128,), jnp.float32),
            pltpu.VMEM((128,), jnp.float32),
            pltpu.VMEM((128,), jnp.float32),
            pltpu.VMEM((128, 16), jnp.float32),
            pltpu.VMEM((tail_n,), jnp.int32),
            pltpu.VMEM((tail_n,), jnp.int32),
            pltpu.VMEM((tail_n,), jnp.int32),
            pltpu.VMEM((tail_n,), jnp.float32),
            pltpu.VMEM((tail_n,), jnp.float32),
            pltpu.VMEM((tail_n,), jnp.float32),
            pltpu.VMEM((tail_n, 16), jnp.float32),
            pltpu.VMEM((16,), jnp.float32),
            pltpu.VMEM((zc, 16), jnp.float32),
            pltpu.VMEM((z1,), jnp.float32),
            pltpu.VMEM_SHARED((n, 16), jnp.float32),
            pltpu.VMEM_SHARED((n,), jnp.float32),
            pltpu.VMEM_SHARED((n,), jnp.float32),
            pltpu.VMEM_SHARED((n,), jnp.float32),
            pltpu.SemaphoreType.DMA,
            pltpu.SemaphoreType.DMA,
            pltpu.SemaphoreType.DMA,
            pltpu.SemaphoreType.DMA,
            pltpu.SemaphoreType.DMA,
            pltpu.SemaphoreType.DMA,
            pltpu.SemaphoreType.DMA,
        ],
        compiler_params=pltpu.CompilerParams(use_tc_tiling_on_sc=False),
    )
    return run(src, dst, table, asn, adn, shift8)


def _shift_arrays(ma, md):
    c = ma[0, 0] + md[0, 0]
    s = jnp.where(c > 0, c, 0.2 * c)
    return s.reshape(1, 1), jnp.broadcast_to(s.reshape(1), (16,))


def kernel(x, edge_index, edge_attr, W1, att_src1, att_dst1, b1,
           W2, att_src2, att_dst2, b2, W_out, b_out):
    n = x.shape[0]
    src = edge_index[0]
    dst = edge_index[1]

    hl1, as1, ad1, ma1, md1 = _prep1(
        x, W1, att_src1.reshape(1, 16), att_dst1.reshape(1, 16))
    sh1_tc, sh1_sc = _shift_arrays(ma1, md1)

    acc1, ssum1 = _edge_pass(
        src, dst, hl1, as1.reshape(n), ad1.reshape(n), sh1_sc, n,
        channel_split=False)

    hl2, as2, ad2, ma2, md2 = _mid(
        acc1.reshape(2, n, 16), ssum1.reshape(2, n, 1), hl1, as1, ad1,
        sh1_tc, b1.reshape(1, 16), W2,
        att_src2.reshape(1, 32), att_dst2.reshape(1, 32))
    sh2_tc, sh2_sc = _shift_arrays(ma2, md2)

    acc2, ssum2 = _edge_pass(
        src, dst, hl2.reshape(2 * n, 16), as2.reshape(n), ad2.reshape(n),
        sh2_sc, n, channel_split=True)

    return _fin(acc2.reshape(2, n, 16), ssum2[:n].reshape(n, 1), hl2,
                as2, ad2, sh2_tc, b2.reshape(1, 32), W_out,
                b_out.reshape(1, 4))
